# baseline (device time: 1550804 ns/iter reference)
import jax
import jax.numpy as jnp
from jax import lax
from jax.experimental import pallas as pl
from jax.experimental.pallas import tpu as pltpu

P = 16
M = 8192
K_PER = 512
N = 4096
QN = N // 4
CHUNK = M // P
N_HOPS = 2 * (P - 1)
NSLOT = 3

_MESH = pl.DeviceIdType.MESH

_RINGS = ((0, +1), (QN, +1), (2 * QN, -1), (3 * QN, -1))


def _fused(x, w):
    def body(x_ref, w_ref, o_ref,
             comm0, comm1, comm2, comm3, tmp_a, tmp_b,
             xsa, xsb, xinit,
             send0, recv0, send1, recv1, send2, recv2, send3, recv3,
             xsem_i, xsem_a, xsem_b,
             osem0, osem1, osem2, osem3,
             credit0, credit1, credit2, credit3):
        my = lax.axis_index("i")
        left = lax.rem(my - 1 + P, P)
        right = lax.rem(my + 1, P)

        barrier = pltpu.get_barrier_semaphore()
        for nbr in (left, right):
            pl.semaphore_signal(barrier, inc=1, device_id=(nbr,),
                                device_id_type=_MESH)
        pl.semaphore_wait(barrier, 2)

        def rows(c):
            return pl.ds(lax.rem(c + 2 * P, P) * CHUNK, CHUNK)

        comms = (comm0, comm1, comm2, comm3)
        sends = (send0, send1, send2, send3)
        recvs = (recv0, recv1, recv2, recv3)
        osems = (osem0, osem1, osem2, osem3)
        credits = (credit0, credit1, credit2, credit3)
        targets = lambda d: right if d > 0 else left
        creditee = lambda d: left if d > 0 else right

        w_a = w_ref[:, pl.ds(0, 2 * QN)]
        w_b = w_ref[:, pl.ds(2 * QN, 2 * QN)]

        ic = pltpu.make_async_copy(x_ref.at[rows(my)], xinit, xsem_i)
        ic.start()
        pend_a = pltpu.make_async_copy(x_ref.at[rows(my - 1)], xsa.at[0],
                                       xsem_a)
        pend_b = pltpu.make_async_copy(x_ref.at[rows(my + 1)], xsb.at[0],
                                       xsem_b)
        pend_a.start()
        pend_b.start()
        ic.wait()
        init_a = jnp.dot(xinit[...], w_a, preferred_element_type=jnp.float32)
        init_b = jnp.dot(xinit[...], w_b, preferred_element_type=jnp.float32)
        comm0[0] = init_a[:, :QN]
        comm1[0] = init_a[:, QN:]
        comm2[0] = init_b[:, :QN]
        comm3[0] = init_b[:, QN:]

        def tmp_slice(i):
            half = tmp_a if i < 2 else tmp_b
            off = (i % 2) * QN
            return half[:, pl.ds(off, QN)]

        prev_out = []
        for h in range(N_HOPS):
            s = h % NSLOT
            r = (h + 1) % NSLOT

            if h >= 2:
                for cr in credits:
                    pl.semaphore_wait(cr, 1)

            rds = []
            for i in (0, 2, 1, 3):
                col, d = _RINGS[i]
                rd = pltpu.make_async_remote_copy(
                    src_ref=comms[i].at[s], dst_ref=comms[i].at[r],
                    send_sem=sends[i].at[s], recv_sem=recvs[i].at[r],
                    device_id=(targets(d),), device_id_type=_MESH,
                )
                rd.start()
                rds.append((i, rd))
            rds.sort()
            rd_by_ring = dict(rds)

            if h < P - 1:
                xs = h % 2
                pend_a.wait()
                pend_b.wait()
                if h + 1 < P - 1:
                    pend_a = pltpu.make_async_copy(
                        x_ref.at[rows(my - h - 2)], xsa.at[(h + 1) % 2],
                        xsem_a)
                    pend_b = pltpu.make_async_copy(
                        x_ref.at[rows(my + h + 2)], xsb.at[(h + 1) % 2],
                        xsem_b)
                    pend_a.start()
                    pend_b.start()
                tmp_a[...] = jnp.dot(xsa[xs], w_a,
                                     preferred_element_type=jnp.float32)
                tmp_b[...] = jnp.dot(xsb[xs], w_b,
                                     preferred_element_type=jnp.float32)

            new_out = []
            if h < P - 1:
                for pair in ((0, 2), (1, 3)):
                    for i in pair:
                        rd_by_ring[i].wait()
                    for i in pair:
                        comms[i][r] = comms[i][r] + tmp_slice(i)
                if h == P - 2:
                    for i in range(4):
                        col, d = _RINGS[i]
                        oc = pltpu.make_async_copy(
                            comms[i].at[r],
                            o_ref.at[rows(my + d), pl.ds(col, QN)],
                            osems[i])
                        oc.start()
                        new_out.append(oc)
            else:
                g = h - (P - 1)
                for pair in ((0, 2), (1, 3)):
                    for i in pair:
                        rd_by_ring[i].wait()
                    for i in pair:
                        col, d = _RINGS[i]
                        oc = pltpu.make_async_copy(
                            comms[i].at[r],
                            o_ref.at[rows(my - d * g), pl.ds(col, QN)],
                            osems[i])
                        oc.start()
                        new_out.append(oc)

            for oc in prev_out:
                oc.wait()
            prev_out = new_out

            if h < N_HOPS - 2:
                for i in range(4):
                    _, d = _RINGS[i]
                    pl.semaphore_signal(credits[i], inc=1,
                                        device_id=(creditee(d),),
                                        device_id_type=_MESH)

        for oc in prev_out:
            oc.wait()

    return pl.pallas_call(
        body,
        out_shape=jax.ShapeDtypeStruct((M, N), jnp.float32),
        in_specs=[
            pl.BlockSpec(memory_space=pl.ANY),
            pl.BlockSpec(memory_space=pltpu.MemorySpace.VMEM),
        ],
        out_specs=pl.BlockSpec(memory_space=pl.ANY),
        scratch_shapes=[
            pltpu.VMEM((NSLOT, CHUNK, QN), jnp.float32),
            pltpu.VMEM((NSLOT, CHUNK, QN), jnp.float32),
            pltpu.VMEM((NSLOT, CHUNK, QN), jnp.float32),
            pltpu.VMEM((NSLOT, CHUNK, QN), jnp.float32),
            pltpu.VMEM((CHUNK, 2 * QN), jnp.float32),
            pltpu.VMEM((CHUNK, 2 * QN), jnp.float32),
            pltpu.VMEM((2, CHUNK, K_PER), jnp.float32),
            pltpu.VMEM((2, CHUNK, K_PER), jnp.float32),
            pltpu.VMEM((CHUNK, K_PER), jnp.float32),
            pltpu.SemaphoreType.DMA((NSLOT,)),
            pltpu.SemaphoreType.DMA((NSLOT,)),
            pltpu.SemaphoreType.DMA((NSLOT,)),
            pltpu.SemaphoreType.DMA((NSLOT,)),
            pltpu.SemaphoreType.DMA((NSLOT,)),
            pltpu.SemaphoreType.DMA((NSLOT,)),
            pltpu.SemaphoreType.DMA((NSLOT,)),
            pltpu.SemaphoreType.DMA((NSLOT,)),
            pltpu.SemaphoreType.DMA,
            pltpu.SemaphoreType.DMA,
            pltpu.SemaphoreType.DMA,
            pltpu.SemaphoreType.DMA,
            pltpu.SemaphoreType.DMA,
            pltpu.SemaphoreType.DMA,
            pltpu.SemaphoreType.DMA,
            pltpu.SemaphoreType.REGULAR,
            pltpu.SemaphoreType.REGULAR,
            pltpu.SemaphoreType.REGULAR,
            pltpu.SemaphoreType.REGULAR,
        ],
        compiler_params=pltpu.CompilerParams(
            collective_id=0, vmem_limit_bytes=100 * 1024 * 1024
        ),
    )(x, w)


def kernel(x, w_mat):
    return _fused(x, w_mat)
